# Initial kernel scaffold; baseline (speedup 1.0000x reference)
import jax
import jax.numpy as jnp
from jax.experimental import pallas as pl


def kernel(x, batch, W, b):
    raise NotImplementedError("write your pallas kernel here")


# speedup vs baseline: 1.1855x; 1.1855x over previous
import functools

import jax
import jax.numpy as jnp
from jax import lax
from jax.experimental import pallas as pl
from jax.experimental.pallas import tpu as pltpu
from jax.experimental.pallas import tpu_sc as plsc

_NUM_SEG = 1024
_N = 100000
_D = 128
_NC = 2
_NS = 16
_NW = _NC * _NS
_SPW = _NUM_SEG // _NW
_R = 256


def _sc_body(x_hbm, batch_hbm, w_hbm, bv_hbm, st_hbm,
             o_sum_hbm, o_max_hbm,
             xb, bb, wv, bvv, stv, tmp, acc_s, acc_m):
  wid = lax.axis_index("s") * _NC + lax.axis_index("c")
  seg_base = wid * _SPW

  pltpu.sync_copy(st_hbm, stv)
  pltpu.sync_copy(w_hbm, wv)
  pltpu.sync_copy(bv_hbm, bvv)

  sv = stv[pl.ds(wid, 16)]
  start = sv[0]
  end = sv[1]

  wk = [wv[pl.ds(16 * k, 16)] for k in range(8)]
  bval = bvv[...]

  zero = jnp.zeros((16,), jnp.float32)
  ninf = jnp.full((16,), -jnp.inf, dtype=jnp.float32)

  def init_body(i, c):
    acc_s[pl.ds(i * 16, 16)] = zero
    acc_m[pl.ds(i * 16, 16)] = ninf
    return c

  lax.fori_loop(0, _SPW * _D // 16, init_body, 0)

  astart = (start // 8) * 8
  nblk = (end - astart + _R - 1) // _R

  def blk_body(j, c):
    intended = astart + j * _R
    blk = jnp.minimum(intended, _N - _R)
    shift = intended - blk
    pltpu.sync_copy(x_hbm.at[pl.ds(blk * _D, _R * _D)], xb)
    pltpu.sync_copy(batch_hbm.at[pl.ds(blk, _R)], bb)

    def grp_body(g, c2):
      base = g * 16
      lv = bb[pl.ds(base, 16)] - seg_base
      for r in range(16):
        row = base + r
        p = [xb[pl.ds(row * _D + 16 * k, 16)] * wk[k] for k in range(8)]
        s = ((p[0] + p[1]) + (p[2] + p[3])) + ((p[4] + p[5]) + (p[6] + p[7]))
        tmp[pl.ds(r * 16, 16)] = s
      iota16 = lax.iota(jnp.int32, 16) * 16
      z = plsc.load_gather(tmp, [iota16])
      for j in range(1, 16):
        z = z + plsc.load_gather(tmp, [iota16 + j])
      sig = 1.0 / (1.0 + jnp.exp(-(z + bval)))
      for r in range(16):
        row = base + r
        loc = lv[r]
        scv = jnp.full((16,), sig[r], dtype=jnp.float32)
        valid = jnp.logical_and(
            jnp.logical_and(loc >= 0, loc < _SPW), base + r >= shift)

        @pl.when(valid)
        def _():
          off = loc * _D
          for k in range(8):
            o = pl.ds(off + 16 * k, 16)
            acc_s[o] = acc_s[o] + scv * xb[pl.ds(row * _D + 16 * k, 16)]
            acc_m[o] = jnp.maximum(acc_m[o], xb[pl.ds(row * _D + 16 * k, 16)])

      return c2

    lax.fori_loop(0, _R // 16, grp_body, 0)
    return c

  lax.fori_loop(0, nblk, blk_body, 0)

  pltpu.sync_copy(acc_s, o_sum_hbm.at[pl.ds(seg_base * _D, _SPW * _D)])
  pltpu.sync_copy(acc_m, o_max_hbm.at[pl.ds(seg_base * _D, _SPW * _D)])


_sc_call = functools.partial(
    pl.kernel,
    out_type=(
        jax.ShapeDtypeStruct((_NUM_SEG * _D,), jnp.float32),
        jax.ShapeDtypeStruct((_NUM_SEG * _D,), jnp.float32),
    ),
    mesh=plsc.VectorSubcoreMesh(core_axis_name="c", subcore_axis_name="s"),
    compiler_params=pltpu.CompilerParams(needs_layout_passes=False),
    scratch_types=[
        pltpu.VMEM((_R * _D,), jnp.float32),
        pltpu.VMEM((_R,), jnp.int32),
        pltpu.VMEM((_D,), jnp.float32),
        pltpu.VMEM((16,), jnp.float32),
        pltpu.VMEM((48,), jnp.int32),
        pltpu.VMEM((256,), jnp.float32),
        pltpu.VMEM((_SPW * _D,), jnp.float32),
        pltpu.VMEM((_SPW * _D,), jnp.float32),
    ],
)(_sc_body)


@jax.jit
def kernel(x, batch, W, b):
  bounds = jnp.arange(0, _NUM_SEG + 1, _SPW, dtype=jnp.int32)
  starts = jnp.searchsorted(batch, bounds, side="left").astype(jnp.int32)
  starts = jnp.concatenate(
      [starts, jnp.zeros((48 - _NW - 1,), jnp.int32)])
  wflat = W.reshape((_D,)).astype(jnp.float32)
  bv = jnp.broadcast_to(b.astype(jnp.float32), (16,))
  o_sum, o_max = _sc_call(x.reshape((_N * _D,)), batch, wflat, bv, starts)
  return jnp.concatenate(
      [o_sum.reshape((_NUM_SEG, _D)), o_max.reshape((_NUM_SEG, _D))], axis=1)


# speedup vs baseline: 2.6260x; 2.2150x over previous
import functools

import jax
import jax.numpy as jnp
from jax import lax
from jax.experimental import pallas as pl
from jax.experimental.pallas import tpu as pltpu
from jax.experimental.pallas import tpu_sc as plsc

_NUM_SEG = 1024
_N = 100000
_D = 128
_NC = 2
_NS = 16
_NW = _NC * _NS
_SPW = _NUM_SEG // _NW
_R = 256


def _sc_body(x_hbm, batch_hbm, w_hbm, bv_hbm, st_hbm,
             o_sum_hbm, o_max_hbm,
             xb, bb, wv, bvv, stv, tmp, acc_s, acc_m):
  wid = lax.axis_index("s") * _NC + lax.axis_index("c")
  seg_base = wid * _SPW

  pltpu.sync_copy(st_hbm, stv)
  pltpu.sync_copy(w_hbm, wv)
  pltpu.sync_copy(bv_hbm, bvv)

  sv = stv[pl.ds(wid, 16)]
  start = sv[0]
  end = sv[1]

  wk = [wv[pl.ds(16 * k, 16)] for k in range(8)]
  bval = bvv[...]

  zero = jnp.zeros((16,), jnp.float32)
  ninf = jnp.full((16,), -jnp.inf, dtype=jnp.float32)

  def init_body(i, c):
    acc_s[pl.ds(i * 16, 16)] = zero
    acc_m[pl.ds(i * 16, 16)] = ninf
    return c

  lax.fori_loop(0, _SPW * _D // 16, init_body, 0)

  astart = (start // 8) * 8
  nblk = (end - astart + _R - 1) // _R

  def blk_body(j, c):
    intended = astart + j * _R
    blk = jnp.minimum(intended, _N - _R)
    shift = intended - blk
    pltpu.sync_copy(x_hbm.at[pl.ds(blk * _D, _R * _D)], xb)
    pltpu.sync_copy(batch_hbm.at[pl.ds(blk, _R)], bb)

    def grp_body(g, c2):
      base = g * 16
      lv = bb[pl.ds(base, 16)] - seg_base
      for r in range(16):
        row = base + r
        p = [xb[pl.ds(row * _D + 16 * k, 16)] * wk[k] for k in range(8)]
        s = ((p[0] + p[1]) + (p[2] + p[3])) + ((p[4] + p[5]) + (p[6] + p[7]))
        tmp[pl.ds(r * 16, 16)] = s
      iota16 = lax.iota(jnp.int32, 16)
      iota16x = iota16 * 16
      z = plsc.load_gather(tmp, [iota16x])
      for j in range(1, 16):
        z = z + plsc.load_gather(tmp, [iota16x + j])
      sig = 1.0 / (1.0 + jnp.exp(-(z + bval)))
      validv = jnp.logical_and(
          jnp.logical_and(lv >= 0, lv < _SPW), iota16 + base >= shift)
      sig_eff = jnp.where(validv, sig, 0.0)
      mgate = jnp.where(validv, 0.0, -jnp.inf)
      locc = jnp.clip(lv, 0, _SPW - 1)
      for r in range(16):
        row = base + r
        off = locc[r] * _D
        scv = jnp.full((16,), sig_eff[r], dtype=jnp.float32)
        mgv = jnp.full((16,), mgate[r], dtype=jnp.float32)
        for k in range(8):
          o = pl.ds(off + 16 * k, 16)
          xk = xb[pl.ds(row * _D + 16 * k, 16)]
          plsc.addupdate(acc_s.at[o], scv * xk)
          acc_m[o] = jnp.maximum(acc_m[o], xk + mgv)

      return c2

    lax.fori_loop(0, _R // 16, grp_body, 0)
    return c

  lax.fori_loop(0, nblk, blk_body, 0)

  pltpu.sync_copy(acc_s, o_sum_hbm.at[pl.ds(seg_base * _D, _SPW * _D)])
  pltpu.sync_copy(acc_m, o_max_hbm.at[pl.ds(seg_base * _D, _SPW * _D)])


_sc_call = functools.partial(
    pl.kernel,
    out_type=(
        jax.ShapeDtypeStruct((_NUM_SEG * _D,), jnp.float32),
        jax.ShapeDtypeStruct((_NUM_SEG * _D,), jnp.float32),
    ),
    mesh=plsc.VectorSubcoreMesh(core_axis_name="c", subcore_axis_name="s"),
    compiler_params=pltpu.CompilerParams(needs_layout_passes=False),
    scratch_types=[
        pltpu.VMEM((_R * _D,), jnp.float32),
        pltpu.VMEM((_R,), jnp.int32),
        pltpu.VMEM((_D,), jnp.float32),
        pltpu.VMEM((16,), jnp.float32),
        pltpu.VMEM((48,), jnp.int32),
        pltpu.VMEM((256,), jnp.float32),
        pltpu.VMEM((_SPW * _D,), jnp.float32),
        pltpu.VMEM((_SPW * _D,), jnp.float32),
    ],
)(_sc_body)


@jax.jit
def kernel(x, batch, W, b):
  bounds = jnp.arange(0, _NUM_SEG + 1, _SPW, dtype=jnp.int32)
  starts = jnp.searchsorted(batch, bounds, side="left").astype(jnp.int32)
  starts = jnp.concatenate(
      [starts, jnp.zeros((48 - _NW - 1,), jnp.int32)])
  wflat = W.reshape((_D,)).astype(jnp.float32)
  bv = jnp.broadcast_to(b.astype(jnp.float32), (16,))
  o_sum, o_max = _sc_call(x.reshape((_N * _D,)), batch, wflat, bv, starts)
  return jnp.concatenate(
      [o_sum.reshape((_NUM_SEG, _D)), o_max.reshape((_NUM_SEG, _D))], axis=1)


# speedup vs baseline: 2.8703x; 1.0930x over previous
import functools

import jax
import jax.numpy as jnp
from jax import lax
from jax.experimental import pallas as pl
from jax.experimental.pallas import tpu as pltpu
from jax.experimental.pallas import tpu_sc as plsc

_NUM_SEG = 1024
_N = 100000
_D = 128
_NC = 2
_NS = 16
_NW = _NC * _NS
_SPW = _NUM_SEG // _NW
_R = 256


def _sc_body(x_hbm, batch_hbm, w_hbm, bv_hbm, st_hbm,
             o_sum_hbm, o_max_hbm,
             xb0, xb1, bb0, bb1, wv, bvv, stv, tmp, acc_s, acc_m,
             sx0, sx1, sb0, sb1):
  wid = lax.axis_index("s") * _NC + lax.axis_index("c")
  seg_base = wid * _SPW

  pltpu.sync_copy(st_hbm, stv)
  pltpu.sync_copy(w_hbm, wv)
  pltpu.sync_copy(bv_hbm, bvv)

  sv = stv[pl.ds(wid, 16)]
  start = sv[0]
  end = sv[1]

  wk = [wv[pl.ds(16 * k, 16)] for k in range(8)]
  bval = bvv[...]

  zero = jnp.zeros((16,), jnp.float32)
  ninf = jnp.full((16,), -jnp.inf, dtype=jnp.float32)

  def init_body(i, c):
    acc_s[pl.ds(i * 16, 16)] = zero
    acc_m[pl.ds(i * 16, 16)] = ninf
    return c

  lax.fori_loop(0, _SPW * _D // 16, init_body, 0)

  astart = (start // 8) * 8
  nblk = (end - astart + _R - 1) // _R

  def blk_of(j):
    return jnp.minimum(astart + j * _R, _N - _R)

  def dma_x(j, buf, sem):
    return pltpu.make_async_copy(
        x_hbm.at[pl.ds(blk_of(j) * _D, _R * _D)], buf, sem)

  def dma_b(j, buf, sem):
    return pltpu.make_async_copy(
        batch_hbm.at[pl.ds(blk_of(j), _R)], buf, sem)

  @pl.when(nblk > 0)
  def _():
    dma_x(0, xb0, sx0).start()
    dma_b(0, bb0, sb0).start()

  def process_block(j, xb, bb):
    intended = astart + j * _R
    blk = blk_of(j)
    shift = intended - blk

    def grp_body(g, c2):
      base = g * 16
      lv = bb[pl.ds(base, 16)] - seg_base
      for r in range(16):
        row = base + r
        p = [xb[pl.ds(row * _D + 16 * k, 16)] * wk[k] for k in range(8)]
        s = ((p[0] + p[1]) + (p[2] + p[3])) + ((p[4] + p[5]) + (p[6] + p[7]))
        tmp[pl.ds(r * 16, 16)] = s
      iota16 = lax.iota(jnp.int32, 16)
      iota16x = iota16 * 16
      z = plsc.load_gather(tmp, [iota16x])
      for j in range(1, 16):
        z = z + plsc.load_gather(tmp, [iota16x + j])
      sig = 1.0 / (1.0 + jnp.exp(-(z + bval)))
      validv = jnp.logical_and(
          jnp.logical_and(lv >= 0, lv < _SPW), iota16 + base >= shift)
      sig_eff = jnp.where(validv, sig, 0.0)
      mgate = jnp.where(validv, 0.0, -jnp.inf)
      locc = jnp.clip(lv, 0, _SPW - 1)
      for r in range(16):
        row = base + r
        off = locc[r] * _D
        scv = jnp.full((16,), sig_eff[r], dtype=jnp.float32)
        mgv = jnp.full((16,), mgate[r], dtype=jnp.float32)
        for k in range(8):
          o = pl.ds(off + 16 * k, 16)
          xk = xb[pl.ds(row * _D + 16 * k, 16)]
          plsc.addupdate(acc_s.at[o], scv * xk)
          acc_m[o] = jnp.maximum(acc_m[o], xk + mgv)

      return c2

    lax.fori_loop(0, _R // 16, grp_body, 0)

  bufs = ((xb0, bb0, sx0, sb0), (xb1, bb1, sx1, sb1))

  def blk2_body(j2, c):
    for p in range(2):
      jj = 2 * j2 + p
      xb, bb, sx, sb = bufs[p]
      nxb, nbb, nsx, nsb = bufs[1 - p]

      @pl.when(jj < nblk)
      def _():
        @pl.when(jj + 1 < nblk)
        def _():
          dma_x(jj + 1, nxb, nsx).start()
          dma_b(jj + 1, nbb, nsb).start()

        dma_x(jj, xb, sx).wait()
        dma_b(jj, bb, sb).wait()
        process_block(jj, xb, bb)

    return c

  lax.fori_loop(0, (nblk + 1) // 2, blk2_body, 0)

  pltpu.sync_copy(acc_s, o_sum_hbm.at[pl.ds(seg_base * _D, _SPW * _D)])
  pltpu.sync_copy(acc_m, o_max_hbm.at[pl.ds(seg_base * _D, _SPW * _D)])


_sc_call = functools.partial(
    pl.kernel,
    out_type=(
        jax.ShapeDtypeStruct((_NUM_SEG * _D,), jnp.float32),
        jax.ShapeDtypeStruct((_NUM_SEG * _D,), jnp.float32),
    ),
    mesh=plsc.VectorSubcoreMesh(core_axis_name="c", subcore_axis_name="s"),
    compiler_params=pltpu.CompilerParams(needs_layout_passes=False),
    scratch_types=[
        pltpu.VMEM((_R * _D,), jnp.float32),
        pltpu.VMEM((_R * _D,), jnp.float32),
        pltpu.VMEM((_R,), jnp.int32),
        pltpu.VMEM((_R,), jnp.int32),
        pltpu.VMEM((_D,), jnp.float32),
        pltpu.VMEM((16,), jnp.float32),
        pltpu.VMEM((48,), jnp.int32),
        pltpu.VMEM((256,), jnp.float32),
        pltpu.VMEM((_SPW * _D,), jnp.float32),
        pltpu.VMEM((_SPW * _D,), jnp.float32),
        pltpu.SemaphoreType.DMA,
        pltpu.SemaphoreType.DMA,
        pltpu.SemaphoreType.DMA,
        pltpu.SemaphoreType.DMA,
    ],
)(_sc_body)


@jax.jit
def kernel(x, batch, W, b):
  bounds = jnp.arange(0, _NUM_SEG + 1, _SPW, dtype=jnp.int32)
  starts = jnp.searchsorted(batch, bounds, side="left").astype(jnp.int32)
  starts = jnp.concatenate(
      [starts, jnp.zeros((48 - _NW - 1,), jnp.int32)])
  wflat = W.reshape((_D,)).astype(jnp.float32)
  bv = jnp.broadcast_to(b.astype(jnp.float32), (16,))
  o_sum, o_max = _sc_call(x.reshape((_N * _D,)), batch, wflat, bv, starts)
  return jnp.concatenate(
      [o_sum.reshape((_NUM_SEG, _D)), o_max.reshape((_NUM_SEG, _D))], axis=1)


# speedup vs baseline: 3.0458x; 1.0612x over previous
import functools

import jax
import jax.numpy as jnp
from jax import lax
from jax.experimental import pallas as pl
from jax.experimental.pallas import tpu as pltpu
from jax.experimental.pallas import tpu_sc as plsc

_NUM_SEG = 1024
_N = 100000
_D = 128
_NC = 2
_NS = 16
_NW = _NC * _NS
_SPW = _NUM_SEG // _NW
_R = 256


def _sc_body(x_hbm, batch_hbm, w_hbm, bv_hbm, st_hbm,
             o_sum_hbm, o_max_hbm,
             xb0, xb1, bb0, bb1, wv, bvv, stv, tmp, acc_s, acc_m,
             sx0, sx1, sb0, sb1):
  wid = lax.axis_index("s") * _NC + lax.axis_index("c")
  seg_base = wid * _SPW

  pltpu.sync_copy(st_hbm, stv)
  pltpu.sync_copy(w_hbm, wv)
  pltpu.sync_copy(bv_hbm, bvv)

  sv = stv[pl.ds(wid, 16)]
  start = sv[0]
  end = sv[1]

  wk = [wv[pl.ds(16 * k, 16)] for k in range(8)]
  bval = bvv[...]

  zero = jnp.zeros((16,), jnp.float32)
  ninf = jnp.full((16,), -jnp.inf, dtype=jnp.float32)

  def init_body(i, c):
    acc_s[pl.ds(i * 16, 16)] = zero
    acc_m[pl.ds(i * 16, 16)] = ninf
    return c

  lax.fori_loop(0, (_SPW + 1) * _D // 16, init_body, 0)

  astart = (start // 8) * 8
  nblk = (end - astart + _R - 1) // _R

  def blk_of(j):
    return jnp.minimum(astart + j * _R, _N - _R)

  def dma_x(j, buf, sem):
    return pltpu.make_async_copy(
        x_hbm.at[pl.ds(blk_of(j) * _D, _R * _D)], buf, sem)

  def dma_b(j, buf, sem):
    return pltpu.make_async_copy(
        batch_hbm.at[pl.ds(blk_of(j), _R)], buf, sem)

  @pl.when(nblk > 0)
  def _():
    dma_x(0, xb0, sx0).start()
    dma_b(0, bb0, sb0).start()

  def process_block(j, xb, bb):
    intended = astart + j * _R
    blk = blk_of(j)
    shift = intended - blk

    def grp_body(g, c2):
      base = g * 16
      lv = bb[pl.ds(base, 16)] - seg_base
      for r in range(16):
        row = base + r
        p = [xb[pl.ds(row * _D + 16 * k, 16)] * wk[k] for k in range(8)]
        s = ((p[0] + p[1]) + (p[2] + p[3])) + ((p[4] + p[5]) + (p[6] + p[7]))
        tmp[pl.ds(r * 16, 16)] = s
      iota16 = lax.iota(jnp.int32, 16)
      iota16x = iota16 * 16
      z = plsc.load_gather(tmp, [iota16x])
      for j in range(1, 16):
        z = z + plsc.load_gather(tmp, [iota16x + j])
      sig = 1.0 / (1.0 + jnp.exp(-(z + bval)))
      validv = jnp.logical_and(
          jnp.logical_and(lv >= 0, lv < _SPW), iota16 + base >= shift)
      locc = jnp.where(validv, lv, _SPW)
      for r in range(16):
        row = base + r
        off = locc[r] * _D
        scv = jnp.full((16,), sig[r], dtype=jnp.float32)
        for k in range(8):
          o = pl.ds(off + 16 * k, 16)
          xk = xb[pl.ds(row * _D + 16 * k, 16)]
          plsc.addupdate(acc_s.at[o], scv * xk)
          acc_m[o] = jnp.maximum(acc_m[o], xk)

      return c2

    lax.fori_loop(0, _R // 16, grp_body, 0)

  bufs = ((xb0, bb0, sx0, sb0), (xb1, bb1, sx1, sb1))

  def blk2_body(j2, c):
    for p in range(2):
      jj = 2 * j2 + p
      xb, bb, sx, sb = bufs[p]
      nxb, nbb, nsx, nsb = bufs[1 - p]

      @pl.when(jj < nblk)
      def _():
        @pl.when(jj + 1 < nblk)
        def _():
          dma_x(jj + 1, nxb, nsx).start()
          dma_b(jj + 1, nbb, nsb).start()

        dma_x(jj, xb, sx).wait()
        dma_b(jj, bb, sb).wait()
        process_block(jj, xb, bb)

    return c

  lax.fori_loop(0, (nblk + 1) // 2, blk2_body, 0)

  pltpu.sync_copy(acc_s.at[pl.ds(0, _SPW * _D)],
                  o_sum_hbm.at[pl.ds(seg_base * _D, _SPW * _D)])
  pltpu.sync_copy(acc_m.at[pl.ds(0, _SPW * _D)],
                  o_max_hbm.at[pl.ds(seg_base * _D, _SPW * _D)])


_sc_call = functools.partial(
    pl.kernel,
    out_type=(
        jax.ShapeDtypeStruct((_NUM_SEG * _D,), jnp.float32),
        jax.ShapeDtypeStruct((_NUM_SEG * _D,), jnp.float32),
    ),
    mesh=plsc.VectorSubcoreMesh(core_axis_name="c", subcore_axis_name="s"),
    compiler_params=pltpu.CompilerParams(needs_layout_passes=False),
    scratch_types=[
        pltpu.VMEM((_R * _D,), jnp.float32),
        pltpu.VMEM((_R * _D,), jnp.float32),
        pltpu.VMEM((_R,), jnp.int32),
        pltpu.VMEM((_R,), jnp.int32),
        pltpu.VMEM((_D,), jnp.float32),
        pltpu.VMEM((16,), jnp.float32),
        pltpu.VMEM((48,), jnp.int32),
        pltpu.VMEM((256,), jnp.float32),
        pltpu.VMEM(((_SPW + 1) * _D,), jnp.float32),
        pltpu.VMEM(((_SPW + 1) * _D,), jnp.float32),
        pltpu.SemaphoreType.DMA,
        pltpu.SemaphoreType.DMA,
        pltpu.SemaphoreType.DMA,
        pltpu.SemaphoreType.DMA,
    ],
)(_sc_body)


@jax.jit
def kernel(x, batch, W, b):
  bounds = jnp.arange(0, _NUM_SEG + 1, _SPW, dtype=jnp.int32)
  starts = jnp.searchsorted(batch, bounds, side="left").astype(jnp.int32)
  starts = jnp.concatenate(
      [starts, jnp.zeros((48 - _NW - 1,), jnp.int32)])
  wflat = W.reshape((_D,)).astype(jnp.float32)
  bv = jnp.broadcast_to(b.astype(jnp.float32), (16,))
  o_sum, o_max = _sc_call(x.reshape((_N * _D,)), batch, wflat, bv, starts)
  return jnp.concatenate(
      [o_sum.reshape((_NUM_SEG, _D)), o_max.reshape((_NUM_SEG, _D))], axis=1)
